# baseline (speedup 1.0000x reference)
import functools

import jax
import jax.numpy as jnp
from jax import lax
from jax.experimental import pallas as pl
from jax.experimental.pallas import tpu as pltpu
from jax.experimental.pallas import tpu_sc as plsc

Q = 256
M = 8192
G = 1024
CN = 128
VOCAB = 16384
EPS = 1e-7
NW = 32
MB = M // NW
L = 16


def _sc_body(ment_hbm, gold_hbm, clus_hbm, out_hbm, table_v, gold_v, clus_v,
             ment_v, out_v, sem):
    wid = lax.axis_index("s") * 2 + lax.axis_index("c")
    base = wid * MB
    cp_g = pltpu.async_copy(gold_hbm, gold_v, sem)
    cp_c = pltpu.async_copy(clus_hbm, clus_v, sem)
    cp_m = pltpu.async_copy(ment_hbm.at[pl.ds(base, MB)], ment_v, sem)

    neg1 = jnp.full((L,), -1, jnp.int32)

    def init_body(i, c):
        table_v[pl.ds(i * L, L)] = neg1
        return c

    lax.fori_loop(0, VOCAB // L, init_body, 0, unroll=8)
    cp_g.wait()
    cp_c.wait()
    cp_m.wait()

    def scat_body(i, c):
        idx = gold_v[pl.ds(i * L, L)]
        val = clus_v[pl.ds(i * L, L)]
        plsc.store_scatter(table_v, [idx], val)
        return c

    lax.fori_loop(0, G // L, scat_body, 0, unroll=4)

    def gath_body(i, c):
        mi = ment_v[pl.ds(i * L, L)]
        out_v[pl.ds(i * L, L)] = plsc.load_gather(table_v, [mi])
        return c

    lax.fori_loop(0, MB // L, gath_body, 0, unroll=4)
    pltpu.sync_copy(out_v, out_hbm.at[pl.ds(base, MB)])


def _sc_cl_of_m(mention_ids, gold_words, cluster_ids):
    mesh = plsc.VectorSubcoreMesh(core_axis_name="c", subcore_axis_name="s")
    k = functools.partial(
        pl.kernel,
        mesh=mesh,
        compiler_params=pltpu.CompilerParams(needs_layout_passes=False),
        out_type=jax.ShapeDtypeStruct((M,), jnp.int32),
        scratch_types=[
            pltpu.VMEM((VOCAB,), jnp.int32),
            pltpu.VMEM((G,), jnp.int32),
            pltpu.VMEM((G,), jnp.int32),
            pltpu.VMEM((MB,), jnp.int32),
            pltpu.VMEM((MB,), jnp.int32),
            pltpu.SemaphoreType.DMA,
        ],
    )(_sc_body)
    return k(mention_ids, gold_words, cluster_ids)


def _tc_body(logits_ref, cl_ref, clus_ref, cim_ref,
             total_ref, coref_ref, junk_ref, ct_ref):
    M1 = M + 1
    nd = logits_ref[:, :M]
    cl = cl_ref[...]
    clus = clus_ref[...]

    p = jnp.clip(nd, EPS, 1.0 - EPS)
    lp = jnp.log(p)
    l1p = jnp.log(1.0 - p)

    ci_m = lax.broadcasted_iota(jnp.int32, (CN, M), 0)
    onehotT = (cl == ci_m).astype(jnp.float32)
    ci_g = lax.broadcasted_iota(jnp.int32, (CN, G), 0)
    onehot2T = (clus == ci_g).astype(jnp.float32)

    nt = (((1,), (1,)), ((), ()))
    AmT = lax.dot_general(onehotT, lp, nt, preferred_element_type=jnp.float32)
    BmT = lax.dot_general(onehotT, l1p, nt, preferred_element_type=jnp.float32)
    ones_m = jnp.ones((1, M), jnp.float32)
    ones_g = jnp.ones((1, G), jnp.float32)
    n_matched = lax.dot_general(onehotT, ones_m, nt,
                                preferred_element_type=jnp.float32)
    cnt = lax.dot_general(onehot2T, ones_g, nt,
                          preferred_element_type=jnp.float32)
    n_unm = cnt - n_matched

    L0 = jnp.float32(jnp.log(jnp.float32(EPS)))
    L1 = jnp.float32(jnp.log1p(jnp.float32(-EPS)))
    AT = AmT + n_unm * L0
    BT = BmT + n_unm * L1
    T1T = jnp.sum(BT, axis=0, keepdims=True)
    costT = -(AT + T1T - BT)

    lane_m1 = lax.broadcasted_iota(jnp.int32, (1, M1), 1)
    w_nd = (lane_m1 < M).astype(jnp.float32)
    w_dm = (lane_m1 == M).astype(jnp.float32)
    logits = logits_ref[...]
    ntf = (((1,), (1,)), ((), ()))
    rowsumT = lax.dot_general(w_nd, logits, ntf,
                              preferred_element_type=jnp.float32)
    dummyT = lax.dot_general(w_dm, logits, ntf,
                             preferred_element_type=jnp.float32)
    matched = (cl >= 0).astype(jnp.float32)
    msumT = lax.dot_general(matched, nd, nt,
                            preferred_element_type=jnp.float32)
    junkT = rowsumT - msumT
    jdT = junkT + dummyT

    num_clusters = jnp.max(clus) + 1
    subiota = lax.broadcasted_iota(jnp.int32, (CN, 1), 0)
    costT = jnp.where(subiota < num_clusters, costT, jnp.inf)
    ct_ref[...] = jnp.transpose(costT)
    jd = jnp.transpose(jdT)
    rowiota1 = lax.broadcasted_iota(jnp.int32, (Q, 1), 0)
    coliota1 = lax.broadcasted_iota(jnp.int32, (1, CN), 1)

    def body(t, carry):
        qmask, acc1, acc2, picked = carry
        active = t < num_clusters
        ct = ct_ref[...]
        c_eff = ct + qmask
        kstar = jnp.argmin(c_eff).astype(jnp.int32)
        gmin = jnp.min(c_eff)
        qstar = lax.shift_right_logical(kstar, 7)
        cstar = jnp.bitwise_and(kstar, CN - 1)
        qsel = rowiota1 == qstar
        csel = coliota1 == cstar
        acc1 = acc1 + jnp.where(active, gmin, 0.0)
        acc2 = acc2 + jnp.where(active,
                                jnp.sum(jnp.where(qsel, jd, 0.0)), 0.0)
        picked = picked + jnp.where(active & qsel, 1.0, 0.0)
        qmask = qmask + jnp.where(active & qsel, jnp.inf, 0.0)
        ct_ref[...] = jnp.where(csel & active, jnp.inf, ct)
        return qmask, acc1, acc2, picked

    init = (jnp.zeros((Q, 1), jnp.float32), jnp.float32(0.0),
            jnp.float32(0.0), jnp.zeros((Q, 1), jnp.float32))
    _, acc1, acc2, picked = lax.fori_loop(0, CN, body, init, unroll=4)

    num_valid = num_clusters.astype(jnp.float32)
    cost_coref = acc1 / (num_valid * G) + acc2 / num_valid
    pj = jnp.clip(jnp.minimum(junkT, 1.0), EPS, 1.0 - EPS)
    pd = jnp.clip(jnp.minimum(dummyT, 1.0), EPS, 1.0 - EPS)
    Jq = -T1T - jnp.log(1.0 - pj) - jnp.log(pd)
    num_junk = jnp.float32(Q) - num_valid
    cost_junk = jnp.sum((1.0 - jnp.transpose(picked)) * Jq) / (num_junk * (G + 2))
    cim = cim_ref[0, 0]
    total = 5.0 * cost_coref + 5.0 * cost_junk + cim
    total_ref[0, 0] = total
    coref_ref[0, 0] = cost_coref
    junk_ref[0, 0] = cost_junk


def _tc_loss(coref_logits, cl_of_m, cluster_ids, cim):
    out_shapes = [jax.ShapeDtypeStruct((1, 1), jnp.float32)] * 3
    return pl.pallas_call(
        _tc_body,
        out_shape=out_shapes,
        out_specs=[pl.BlockSpec(memory_space=pltpu.SMEM)] * 3,
        scratch_shapes=[pltpu.VMEM((Q, CN), jnp.float32)],
    )(coref_logits, cl_of_m, cluster_ids, cim)


def kernel(coref_logits, mention_ids, gold_words, cluster_ids, cost_is_mention):
    mention_ids = mention_ids.astype(jnp.int32)
    gold_words = gold_words.astype(jnp.int32)
    cluster_ids = cluster_ids.astype(jnp.int32)
    cl_of_m = _sc_cl_of_m(mention_ids, gold_words, cluster_ids)
    total, coref, junk = _tc_loss(
        coref_logits,
        cl_of_m.reshape(1, M),
        cluster_ids.reshape(1, G),
        cost_is_mention.reshape(1, 1).astype(jnp.float32),
    )
    return total[0, 0], coref[0, 0], junk[0, 0]


# speedup vs baseline: 1.2451x; 1.2451x over previous
import functools

import jax
import jax.numpy as jnp
from jax import lax
from jax.experimental import pallas as pl
from jax.experimental.pallas import tpu as pltpu
from jax.experimental.pallas import tpu_sc as plsc

Q = 256
M = 8192
G = 1024
CN = 128
VOCAB = 16384
EPS = 1e-7
NC = 1
NW = NC * 16
MB = M // NW
L = 16


def _sc_body(ment_hbm, gold_hbm, clus_hbm, out_hbm, table_v, gold_v, clus_v,
             ment_v, out_v, sem):
    wid = lax.axis_index("s") * NC + lax.axis_index("c")
    base = wid * MB
    cp_g = pltpu.async_copy(gold_hbm, gold_v, sem)
    cp_c = pltpu.async_copy(clus_hbm, clus_v, sem)
    cp_m = pltpu.async_copy(ment_hbm.at[pl.ds(base, MB)], ment_v, sem)

    neg1 = jnp.full((L,), -1, jnp.int32)

    def init_body(i, c):
        table_v[pl.ds(i * L, L)] = neg1
        return c

    lax.fori_loop(0, VOCAB // L, init_body, 0, unroll=8)
    cp_g.wait()
    cp_c.wait()
    cp_m.wait()

    def scat_body(i, c):
        idx = gold_v[pl.ds(i * L, L)]
        val = clus_v[pl.ds(i * L, L)]
        plsc.store_scatter(table_v, [idx], val)
        return c

    lax.fori_loop(0, G // L, scat_body, 0, unroll=4)

    def gath_body(i, c):
        mi = ment_v[pl.ds(i * L, L)]
        out_v[pl.ds(i * L, L)] = plsc.load_gather(table_v, [mi])
        return c

    lax.fori_loop(0, MB // L, gath_body, 0, unroll=4)
    pltpu.sync_copy(out_v, out_hbm.at[pl.ds(base, MB)])


def _sc_cl_of_m(mention_ids, gold_words, cluster_ids):
    mesh = plsc.VectorSubcoreMesh(core_axis_name="c", subcore_axis_name="s",
                                  num_cores=NC)
    k = functools.partial(
        pl.kernel,
        mesh=mesh,
        compiler_params=pltpu.CompilerParams(needs_layout_passes=False),
        out_type=jax.ShapeDtypeStruct((M,), jnp.int32),
        scratch_types=[
            pltpu.VMEM((VOCAB,), jnp.int32),
            pltpu.VMEM((G,), jnp.int32),
            pltpu.VMEM((G,), jnp.int32),
            pltpu.VMEM((MB,), jnp.int32),
            pltpu.VMEM((MB,), jnp.int32),
            pltpu.SemaphoreType.DMA,
        ],
    )(_sc_body)
    return k(mention_ids, gold_words, cluster_ids)


def _tc_body(logits_ref, cl_ref, clus_ref, cim_ref,
             total_ref, coref_ref, junk_ref):
    M1 = M + 1
    nd = logits_ref[:, :M]
    cl = cl_ref[...]
    clus = clus_ref[...]

    p = jnp.clip(nd, EPS, 1.0 - EPS)
    lp = jnp.log(p)
    l1p = jnp.log(1.0 - p)

    ci_m = lax.broadcasted_iota(jnp.int32, (CN, M), 0)
    onehotT = (cl == ci_m).astype(jnp.float32)
    ci_g = lax.broadcasted_iota(jnp.int32, (CN, G), 0)
    onehot2T = (clus == ci_g).astype(jnp.float32)

    nt = (((1,), (1,)), ((), ()))
    AmT = lax.dot_general(onehotT, lp, nt, preferred_element_type=jnp.float32)
    BmT = lax.dot_general(onehotT, l1p, nt, preferred_element_type=jnp.float32)
    ones_m = jnp.ones((1, M), jnp.float32)
    ones_g = jnp.ones((1, G), jnp.float32)
    n_matched = lax.dot_general(onehotT, ones_m, nt,
                                preferred_element_type=jnp.float32)
    cnt = lax.dot_general(onehot2T, ones_g, nt,
                          preferred_element_type=jnp.float32)
    n_unm = cnt - n_matched

    L0 = jnp.float32(jnp.log(jnp.float32(EPS)))
    L1 = jnp.float32(jnp.log1p(jnp.float32(-EPS)))
    AT = AmT + n_unm * L0
    BT = BmT + n_unm * L1
    T1T = jnp.sum(BT, axis=0, keepdims=True)
    costT = -(AT + T1T - BT)

    lane_m1 = lax.broadcasted_iota(jnp.int32, (1, M1), 1)
    w_nd = (lane_m1 < M).astype(jnp.float32)
    w_dm = (lane_m1 == M).astype(jnp.float32)
    logits = logits_ref[...]
    ntf = (((1,), (1,)), ((), ()))
    rowsumT = lax.dot_general(w_nd, logits, ntf,
                              preferred_element_type=jnp.float32)
    dummyT = lax.dot_general(w_dm, logits, ntf,
                             preferred_element_type=jnp.float32)
    matched = (cl >= 0).astype(jnp.float32)
    msumT = lax.dot_general(matched, nd, nt,
                            preferred_element_type=jnp.float32)
    junkT = rowsumT - msumT
    jdT = junkT + dummyT

    num_clusters = jnp.max(clus) + 1
    subiota = lax.broadcasted_iota(jnp.int32, (CN, 1), 0)
    laneQ = lax.broadcasted_iota(jnp.int32, (1, Q), 1)
    key = (lax.broadcasted_iota(jnp.int32, (CN, Q), 1) * CN
           + lax.broadcasted_iota(jnp.int32, (CN, Q), 0))
    costT = jnp.where(subiota < num_clusters, costT, jnp.inf)
    BIGI = jnp.int32(Q * CN)

    def body(t, carry):
        cT, acc1, acc2, picked = carry
        active = t < num_clusters
        gmin = jnp.min(cT)
        kstar = jnp.min(jnp.where(cT == gmin, key, BIGI))
        qstar = lax.shift_right_logical(kstar, 7)
        cstar = jnp.bitwise_and(kstar, CN - 1)
        qsel = laneQ == qstar
        csel = subiota == cstar
        acc1 = acc1 + jnp.where(active, gmin, 0.0)
        acc2 = acc2 + jnp.where(active,
                                jnp.sum(jnp.where(qsel, jdT, 0.0)), 0.0)
        picked = picked + jnp.where(active & qsel, 1.0, 0.0)
        cT = jnp.where((qsel | csel) & active, jnp.inf, cT)
        return cT, acc1, acc2, picked

    init = (costT, jnp.float32(0.0), jnp.float32(0.0),
            jnp.zeros((1, Q), jnp.float32))
    _, acc1, acc2, picked = lax.fori_loop(0, CN, body, init, unroll=4)

    num_valid = num_clusters.astype(jnp.float32)
    cost_coref = acc1 / (num_valid * G) + acc2 / num_valid
    pj = jnp.clip(jnp.minimum(junkT, 1.0), EPS, 1.0 - EPS)
    pd = jnp.clip(jnp.minimum(dummyT, 1.0), EPS, 1.0 - EPS)
    Jq = -T1T - jnp.log(1.0 - pj) - jnp.log(pd)
    num_junk = jnp.float32(Q) - num_valid
    cost_junk = jnp.sum((1.0 - picked) * Jq) / (num_junk * (G + 2))
    cim = cim_ref[0, 0]
    total = 5.0 * cost_coref + 5.0 * cost_junk + cim
    total_ref[0, 0] = total
    coref_ref[0, 0] = cost_coref
    junk_ref[0, 0] = cost_junk


def _tc_loss(coref_logits, cl_of_m, cluster_ids, cim):
    out_shapes = [jax.ShapeDtypeStruct((1, 1), jnp.float32)] * 3
    return pl.pallas_call(
        _tc_body,
        out_shape=out_shapes,
        out_specs=[pl.BlockSpec(memory_space=pltpu.SMEM)] * 3,
    )(coref_logits, cl_of_m, cluster_ids, cim)


def kernel(coref_logits, mention_ids, gold_words, cluster_ids, cost_is_mention):
    mention_ids = mention_ids.astype(jnp.int32)
    gold_words = gold_words.astype(jnp.int32)
    cluster_ids = cluster_ids.astype(jnp.int32)
    cl_of_m = _sc_cl_of_m(mention_ids, gold_words, cluster_ids)
    total, coref, junk = _tc_loss(
        coref_logits,
        cl_of_m.reshape(1, M),
        cluster_ids.reshape(1, G),
        cost_is_mention.reshape(1, 1).astype(jnp.float32),
    )
    return total[0, 0], coref[0, 0], junk[0, 0]


# speedup vs baseline: 1.4286x; 1.1474x over previous
import functools

import jax
import jax.numpy as jnp
from jax import lax
from jax.experimental import pallas as pl
from jax.experimental.pallas import tpu as pltpu
from jax.experimental.pallas import tpu_sc as plsc

Q = 256
M = 8192
G = 1024
CN = 128
VOCAB = 16384
EPS = 1e-7
NC = 1
NW = NC * 16
MB = M // NW
L = 16


def _sc_body(ment_hbm, gold_hbm, clus_hbm, out_hbm, table_v, gold_v, clus_v,
             ment_v, out_v, sem):
    wid = lax.axis_index("s") * NC + lax.axis_index("c")
    base = wid * MB
    cp_g = pltpu.async_copy(gold_hbm, gold_v, sem)
    cp_c = pltpu.async_copy(clus_hbm, clus_v, sem)
    cp_m = pltpu.async_copy(ment_hbm.at[pl.ds(base, MB)], ment_v, sem)

    neg1 = jnp.full((L,), -1, jnp.int32)

    def init_body(i, c):
        table_v[pl.ds(i * L, L)] = neg1
        return c

    lax.fori_loop(0, VOCAB // L, init_body, 0, unroll=8)
    cp_g.wait()
    cp_c.wait()
    cp_m.wait()

    def scat_body(i, c):
        idx = gold_v[pl.ds(i * L, L)]
        val = clus_v[pl.ds(i * L, L)]
        plsc.store_scatter(table_v, [idx], val)
        return c

    lax.fori_loop(0, G // L, scat_body, 0, unroll=4)

    def gath_body(i, c):
        mi = ment_v[pl.ds(i * L, L)]
        out_v[pl.ds(i * L, L)] = plsc.load_gather(table_v, [mi])
        return c

    lax.fori_loop(0, MB // L, gath_body, 0, unroll=4)
    pltpu.sync_copy(out_v, out_hbm.at[pl.ds(base, MB)])


def _sc_cl_of_m(mention_ids, gold_words, cluster_ids):
    mesh = plsc.VectorSubcoreMesh(core_axis_name="c", subcore_axis_name="s",
                                  num_cores=NC)
    k = functools.partial(
        pl.kernel,
        mesh=mesh,
        compiler_params=pltpu.CompilerParams(needs_layout_passes=False),
        out_type=jax.ShapeDtypeStruct((M,), jnp.int32),
        scratch_types=[
            pltpu.VMEM((VOCAB,), jnp.int32),
            pltpu.VMEM((G,), jnp.int32),
            pltpu.VMEM((G,), jnp.int32),
            pltpu.VMEM((MB,), jnp.int32),
            pltpu.VMEM((MB,), jnp.int32),
            pltpu.SemaphoreType.DMA,
        ],
    )(_sc_body)
    return k(mention_ids, gold_words, cluster_ids)


def _tc_body(logits_ref, cl_ref, clus_ref, cim_ref,
             total_ref, coref_ref, junk_ref):
    M1 = M + 1
    nd = logits_ref[:, :M]
    cl = cl_ref[...]
    clus = clus_ref[...]

    p = jnp.clip(nd, EPS, 1.0 - EPS)
    lp = jnp.log(p)
    l1p = jnp.log(1.0 - p)

    ci_m = lax.broadcasted_iota(jnp.int32, (CN, M), 0)
    onehotT = (cl == ci_m).astype(jnp.float32)
    ci_g = lax.broadcasted_iota(jnp.int32, (CN, G), 0)
    onehot2T = (clus == ci_g).astype(jnp.float32)

    nt = (((1,), (1,)), ((), ()))
    AmT = lax.dot_general(onehotT, lp, nt, preferred_element_type=jnp.float32)
    BmT = lax.dot_general(onehotT, l1p, nt, preferred_element_type=jnp.float32)
    ones_m = jnp.ones((1, M), jnp.float32)
    ones_g = jnp.ones((1, G), jnp.float32)
    n_matched = lax.dot_general(onehotT, ones_m, nt,
                                preferred_element_type=jnp.float32)
    cnt = lax.dot_general(onehot2T, ones_g, nt,
                          preferred_element_type=jnp.float32)
    n_unm = cnt - n_matched

    L0 = jnp.float32(jnp.log(jnp.float32(EPS)))
    L1 = jnp.float32(jnp.log1p(jnp.float32(-EPS)))
    AT = AmT + n_unm * L0
    BT = BmT + n_unm * L1
    T1T = jnp.sum(BT, axis=0, keepdims=True)
    costT = -(AT + T1T - BT)

    lane_m1 = lax.broadcasted_iota(jnp.int32, (1, M1), 1)
    w_nd = (lane_m1 < M).astype(jnp.float32)
    w_dm = (lane_m1 == M).astype(jnp.float32)
    logits = logits_ref[...]
    ntf = (((1,), (1,)), ((), ()))
    rowsumT = lax.dot_general(w_nd, logits, ntf,
                              preferred_element_type=jnp.float32)
    dummyT = lax.dot_general(w_dm, logits, ntf,
                             preferred_element_type=jnp.float32)
    matched = (cl >= 0).astype(jnp.float32)
    msumT = lax.dot_general(matched, nd, nt,
                            preferred_element_type=jnp.float32)
    junkT = rowsumT - msumT
    jdT = junkT + dummyT

    num_clusters = jnp.max(clus) + 1
    subiota = lax.broadcasted_iota(jnp.int32, (CN, 1), 0)
    laneQ = lax.broadcasted_iota(jnp.int32, (1, Q), 1)
    key = (lax.broadcasted_iota(jnp.int32, (CN, Q), 1) * CN
           + lax.broadcasted_iota(jnp.int32, (CN, Q), 0))
    costT = jnp.where(subiota < num_clusters, costT, jnp.inf)
    BIGI = jnp.int32(Q * CN)

    def body(t, carry):
        cT, acc1, acc2, picked = carry
        active = t < num_clusters
        gmin = jnp.min(jnp.min(cT, axis=0, keepdims=True),
                       axis=1, keepdims=True)
        kmask = jnp.where(cT == gmin, key, BIGI)
        kstar = jnp.min(jnp.min(kmask, axis=0, keepdims=True),
                        axis=1, keepdims=True)
        qsel = laneQ == lax.shift_right_logical(kstar, 7)
        csel = subiota == jnp.bitwise_and(kstar, CN - 1)
        acc1 = acc1 + jnp.where(active, gmin, 0.0)
        acc2 = acc2 + jnp.where(active,
                                jnp.sum(jnp.where(qsel, jdT, 0.0),
                                        axis=1, keepdims=True), 0.0)
        picked = picked + jnp.where(active & qsel, 1.0, 0.0)
        cT = jnp.where((qsel | csel) & active, jnp.inf, cT)
        return cT, acc1, acc2, picked

    init = (costT, jnp.zeros((1, 1), jnp.float32), jnp.zeros((1, 1), jnp.float32),
            jnp.zeros((1, Q), jnp.float32))
    _, acc1, acc2, picked = lax.fori_loop(0, CN, body, init, unroll=4)
    acc1 = acc1[0, 0]
    acc2 = acc2[0, 0]

    num_valid = num_clusters.astype(jnp.float32)
    cost_coref = acc1 / (num_valid * G) + acc2 / num_valid
    pj = jnp.clip(jnp.minimum(junkT, 1.0), EPS, 1.0 - EPS)
    pd = jnp.clip(jnp.minimum(dummyT, 1.0), EPS, 1.0 - EPS)
    Jq = -T1T - jnp.log(1.0 - pj) - jnp.log(pd)
    num_junk = jnp.float32(Q) - num_valid
    cost_junk = jnp.sum((1.0 - picked) * Jq) / (num_junk * (G + 2))
    cim = cim_ref[0, 0]
    total = 5.0 * cost_coref + 5.0 * cost_junk + cim
    total_ref[0, 0] = total
    coref_ref[0, 0] = cost_coref
    junk_ref[0, 0] = cost_junk


def _tc_loss(coref_logits, cl_of_m, cluster_ids, cim):
    out_shapes = [jax.ShapeDtypeStruct((1, 1), jnp.float32)] * 3
    return pl.pallas_call(
        _tc_body,
        out_shape=out_shapes,
        out_specs=[pl.BlockSpec(memory_space=pltpu.SMEM)] * 3,
    )(coref_logits, cl_of_m, cluster_ids, cim)


def kernel(coref_logits, mention_ids, gold_words, cluster_ids, cost_is_mention):
    mention_ids = mention_ids.astype(jnp.int32)
    gold_words = gold_words.astype(jnp.int32)
    cluster_ids = cluster_ids.astype(jnp.int32)
    cl_of_m = _sc_cl_of_m(mention_ids, gold_words, cluster_ids)
    total, coref, junk = _tc_loss(
        coref_logits,
        cl_of_m.reshape(1, M),
        cluster_ids.reshape(1, G),
        cost_is_mention.reshape(1, 1).astype(jnp.float32),
    )
    return total[0, 0], coref[0, 0], junk[0, 0]


# speedup vs baseline: 1.4404x; 1.0083x over previous
import functools

import jax
import jax.numpy as jnp
from jax import lax
from jax.experimental import pallas as pl
from jax.experimental.pallas import tpu as pltpu
from jax.experimental.pallas import tpu_sc as plsc

Q = 256
M = 8192
G = 1024
CN = 128
VOCAB = 16384
EPS = 1e-7
NC = 1
NW = NC * 16
MB = M // NW
L = 16


def _sc_body(ment_hbm, gold_hbm, clus_hbm, out_hbm, table_v, gold_v, clus_v,
             ment_v, out_v, sem):
    wid = lax.axis_index("s") * NC + lax.axis_index("c")
    base = wid * MB
    cp_g = pltpu.async_copy(gold_hbm, gold_v, sem)
    cp_c = pltpu.async_copy(clus_hbm, clus_v, sem)
    cp_m = pltpu.async_copy(ment_hbm.at[pl.ds(base, MB)], ment_v, sem)

    neg1 = jnp.full((L,), -1, jnp.int32)

    def init_body(i, c):
        table_v[pl.ds(i * L, L)] = neg1
        return c

    lax.fori_loop(0, VOCAB // L, init_body, 0, unroll=8)
    cp_g.wait()
    cp_c.wait()
    cp_m.wait()

    def scat_body(i, c):
        idx = gold_v[pl.ds(i * L, L)]
        val = clus_v[pl.ds(i * L, L)]
        plsc.store_scatter(table_v, [idx], val)
        return c

    lax.fori_loop(0, G // L, scat_body, 0, unroll=4)

    def gath_body(i, c):
        mi = ment_v[pl.ds(i * L, L)]
        out_v[pl.ds(i * L, L)] = plsc.load_gather(table_v, [mi])
        return c

    lax.fori_loop(0, MB // L, gath_body, 0, unroll=4)
    pltpu.sync_copy(out_v, out_hbm.at[pl.ds(base, MB)])


def _sc_cl_of_m(mention_ids, gold_words, cluster_ids):
    mesh = plsc.VectorSubcoreMesh(core_axis_name="c", subcore_axis_name="s",
                                  num_cores=NC)
    k = functools.partial(
        pl.kernel,
        mesh=mesh,
        compiler_params=pltpu.CompilerParams(needs_layout_passes=False),
        out_type=jax.ShapeDtypeStruct((M,), jnp.int32),
        scratch_types=[
            pltpu.VMEM((VOCAB,), jnp.int32),
            pltpu.VMEM((G,), jnp.int32),
            pltpu.VMEM((G,), jnp.int32),
            pltpu.VMEM((MB,), jnp.int32),
            pltpu.VMEM((MB,), jnp.int32),
            pltpu.SemaphoreType.DMA,
        ],
    )(_sc_body)
    return k(mention_ids, gold_words, cluster_ids)


def _tc_body(logits_ref, cl_ref, clus_ref, cim_ref,
             total_ref, coref_ref, junk_ref):
    M1 = M + 1
    nd = logits_ref[:, :M]
    cl = cl_ref[...]
    clus = clus_ref[...]

    p = jnp.clip(nd, EPS, 1.0 - EPS)
    lp = jnp.log(p)
    l1p = jnp.log(1.0 - p)

    ci_m = lax.broadcasted_iota(jnp.int32, (CN, M), 0)
    onehotT = (cl == ci_m).astype(jnp.float32)
    ci_g = lax.broadcasted_iota(jnp.int32, (CN, G), 0)
    onehot2T = (clus == ci_g).astype(jnp.float32)

    nt = (((1,), (1,)), ((), ()))
    AmT = lax.dot_general(onehotT, lp, nt, preferred_element_type=jnp.float32)
    BmT = lax.dot_general(onehotT, l1p, nt, preferred_element_type=jnp.float32)
    ones_m = jnp.ones((1, M), jnp.float32)
    ones_g = jnp.ones((1, G), jnp.float32)
    n_matched = lax.dot_general(onehotT, ones_m, nt,
                                preferred_element_type=jnp.float32)
    cnt = lax.dot_general(onehot2T, ones_g, nt,
                          preferred_element_type=jnp.float32)
    n_unm = cnt - n_matched

    L0 = jnp.float32(jnp.log(jnp.float32(EPS)))
    L1 = jnp.float32(jnp.log1p(jnp.float32(-EPS)))
    AT = AmT + n_unm * L0
    BT = BmT + n_unm * L1
    T1T = jnp.sum(BT, axis=0, keepdims=True)
    costT = -(AT + T1T - BT)

    lane_m1 = lax.broadcasted_iota(jnp.int32, (1, M1), 1)
    w_nd = (lane_m1 < M).astype(jnp.float32)
    w_dm = (lane_m1 == M).astype(jnp.float32)
    logits = logits_ref[...]
    ntf = (((1,), (1,)), ((), ()))
    rowsumT = lax.dot_general(w_nd, logits, ntf,
                              preferred_element_type=jnp.float32)
    dummyT = lax.dot_general(w_dm, logits, ntf,
                             preferred_element_type=jnp.float32)
    matched = (cl >= 0).astype(jnp.float32)
    msumT = lax.dot_general(matched, nd, nt,
                            preferred_element_type=jnp.float32)
    junkT = rowsumT - msumT
    jdT = junkT + dummyT

    num_clusters = jnp.max(clus) + 1
    subiota = lax.broadcasted_iota(jnp.int32, (CN, 1), 0)
    laneQ = lax.broadcasted_iota(jnp.int32, (1, Q), 1)
    key = (lax.broadcasted_iota(jnp.int32, (CN, Q), 1) * CN
           + lax.broadcasted_iota(jnp.int32, (CN, Q), 0))
    costT = jnp.where(subiota < num_clusters, costT, jnp.inf)
    BIGI = jnp.int32(Q * CN)

    def body(t, carry):
        cT, acc1, acc2, picked = carry
        active = t < num_clusters
        gmin = jnp.min(jnp.min(cT, axis=0, keepdims=True),
                       axis=1, keepdims=True)
        kmask = jnp.where(cT == gmin, key, BIGI)
        kstar = jnp.min(jnp.min(kmask, axis=0, keepdims=True),
                        axis=1, keepdims=True)
        qsel = laneQ == lax.shift_right_logical(kstar, 7)
        csel = subiota == jnp.bitwise_and(kstar, CN - 1)
        acc1 = acc1 + jnp.where(active, gmin, 0.0)
        acc2 = acc2 + jnp.where(active,
                                jnp.sum(jnp.where(qsel, jdT, 0.0),
                                        axis=1, keepdims=True), 0.0)
        picked = picked + jnp.where(active & qsel, 1.0, 0.0)
        cT = jnp.where((qsel | csel) & active, jnp.inf, cT)
        return cT, acc1, acc2, picked

    init = (costT, jnp.zeros((1, 1), jnp.float32), jnp.zeros((1, 1), jnp.float32),
            jnp.zeros((1, Q), jnp.float32))
    _, acc1, acc2, picked = lax.fori_loop(0, CN, body, init, unroll=8)
    acc1 = acc1[0, 0]
    acc2 = acc2[0, 0]

    num_valid = num_clusters.astype(jnp.float32)
    cost_coref = acc1 / (num_valid * G) + acc2 / num_valid
    pj = jnp.clip(jnp.minimum(junkT, 1.0), EPS, 1.0 - EPS)
    pd = jnp.clip(jnp.minimum(dummyT, 1.0), EPS, 1.0 - EPS)
    Jq = -T1T - jnp.log(1.0 - pj) - jnp.log(pd)
    num_junk = jnp.float32(Q) - num_valid
    cost_junk = jnp.sum((1.0 - picked) * Jq) / (num_junk * (G + 2))
    cim = cim_ref[0, 0]
    total = 5.0 * cost_coref + 5.0 * cost_junk + cim
    total_ref[0, 0] = total
    coref_ref[0, 0] = cost_coref
    junk_ref[0, 0] = cost_junk


def _tc_loss(coref_logits, cl_of_m, cluster_ids, cim):
    out_shapes = [jax.ShapeDtypeStruct((1, 1), jnp.float32)] * 3
    return pl.pallas_call(
        _tc_body,
        out_shape=out_shapes,
        out_specs=[pl.BlockSpec(memory_space=pltpu.SMEM)] * 3,
    )(coref_logits, cl_of_m, cluster_ids, cim)


def kernel(coref_logits, mention_ids, gold_words, cluster_ids, cost_is_mention):
    mention_ids = mention_ids.astype(jnp.int32)
    gold_words = gold_words.astype(jnp.int32)
    cluster_ids = cluster_ids.astype(jnp.int32)
    cl_of_m = _sc_cl_of_m(mention_ids, gold_words, cluster_ids)
    total, coref, junk = _tc_loss(
        coref_logits,
        cl_of_m.reshape(1, M),
        cluster_ids.reshape(1, G),
        cost_is_mention.reshape(1, 1).astype(jnp.float32),
    )
    return total[0, 0], coref[0, 0], junk[0, 0]
